# baseline (speedup 1.0000x reference)
import functools

import jax
import jax.numpy as jnp
from jax import lax
from jax.experimental import pallas as pl
from jax.experimental.pallas import tpu as pltpu
from jax.experimental.pallas import tpu_sc as plsc

NC = 2
NS = 16
NW = NC * NS
CH = 128
BK = 8
G = 64
F0 = 0.7


def _sc_degree(dsts4, consts16, acc_rows, nb0, nb1):
    nbmax = dsts4.shape[1]
    rpt = acc_rows // NS
    zrep = rpt // CH
    mesh = plsc.VectorSubcoreMesh(core_axis_name="c", subcore_axis_name="s")

    @functools.partial(
        pl.kernel,
        out_type=jax.ShapeDtypeStruct((NC, acc_rows, 16), jnp.float32),
        mesh=mesh,
        scratch_types=[
            pltpu.VMEM((nbmax, BK, CH), jnp.int32),
            pltpu.VMEM((CH, 16), jnp.float32),
            pltpu.VMEM((CH, 16), jnp.float32),
            pltpu.VMEM_SHARED((acc_rows, 16), jnp.float32),
        ],
    )
    def deg_kernel(dsts_hbm, consts_hbm, out_hbm, idx_d, zv, ov, acc):
        c = lax.axis_index("c")
        s = lax.axis_index("s")
        w = s * NC + c
        pltpu.sync_copy(dsts_hbm.at[w], idx_d)
        pltpu.sync_copy(consts_hbm.at[0], zv)
        pltpu.sync_copy(consts_hbm.at[1], ov)
        base = s * rpt
        for r in range(zrep):
            pltpu.sync_copy(zv, acc.at[pl.ds(base + r * CH, CH)])
        plsc.subcore_barrier()

        def body(b, carry):
            for r in range(BK):
                pltpu.sync_copy(ov, acc.at[idx_d.at[b, r]], add=True)
            return carry

        lax.fori_loop(0, lax.select(c == 0, nb0, nb1), body, 0)
        plsc.subcore_barrier()
        pltpu.sync_copy(acc.at[pl.ds(base, rpt)],
                        out_hbm.at[c, pl.ds(base, rpt)])

    return deg_kernel(dsts4, consts16)


def _sc_propagate(y, srcs4, dsts4, zblk, acc_rows, nb0, nb1):
    d = y.shape[1]
    rpt = acc_rows // NS
    zrep = rpt // CH
    mesh = plsc.VectorSubcoreMesh(core_axis_name="c", subcore_axis_name="s")
    assert nb0 % 2 == 0 and nb1 % 2 == 0

    @functools.partial(
        pl.kernel,
        out_type=jax.ShapeDtypeStruct((NC, acc_rows, d), jnp.float32),
        mesh=mesh,
        scratch_types=[
            pltpu.VMEM((BK, CH), jnp.int32),
            pltpu.VMEM((BK, CH), jnp.int32),
            pltpu.VMEM((BK, CH), jnp.int32),
            pltpu.VMEM((BK, CH), jnp.int32),
            pltpu.VMEM((CH, d), jnp.float32),
            pltpu.VMEM((CH, d), jnp.float32),
            pltpu.VMEM_SHARED((acc_rows, d), jnp.float32),
            pltpu.SemaphoreType.DMA,
            pltpu.SemaphoreType.DMA,
            pltpu.SemaphoreType.DMA,
            pltpu.SemaphoreType.DMA,
        ],
    )
    def prop_kernel(y_hbm, srcs_hbm, dsts_hbm, zblk_hbm, out_hbm,
                    sb0, db0, sb1, db1, r0, r1, acc,
                    semi0, semi1, semg0, semg1):
        c = lax.axis_index("c")
        s = lax.axis_index("s")
        w = s * NC + c
        sb = (sb0, sb1)
        db = (db0, db1)
        rr = (r0, r1)
        semi = (semi0, semi1)
        semg = (semg0, semg1)

        pltpu.sync_copy(zblk_hbm, r0)
        base = s * rpt
        for r in range(zrep):
            pltpu.sync_copy(r0, acc.at[pl.ds(base + r * CH, CH)])
        plsc.subcore_barrier()

        def fetch_blk(bidx, buf):
            pltpu.async_copy(srcs_hbm.at[w, bidx], sb[buf], semi[buf])
            pltpu.async_copy(dsts_hbm.at[w, bidx], db[buf], semi[buf])

        def wait_blk(buf):
            pltpu.make_async_copy(srcs_hbm.at[w, 0], sb[buf], semi[buf]).wait()
            pltpu.make_async_copy(dsts_hbm.at[w, 0], db[buf], semi[buf]).wait()

        def gather(bbuf, row, rbuf):
            pltpu.async_copy(y_hbm.at[sb[bbuf].at[row]], rr[rbuf], semg[rbuf])

        def wait_gather(rbuf):
            pltpu.make_async_copy(y_hbm.at[sb0.at[0]], rr[rbuf],
                                  semg[rbuf]).wait()

        @pl.when(c == 0)
        def _pipelined():
            fetch_blk(0, 0)
            fetch_blk(1, 1)
            wait_blk(0)
            gather(0, 0, 0)

            def body(i, carry):
                for p in range(2 * BK):
                    bbuf, row, rbuf = (p // BK) % 2, p % BK, p % 2
                    if p == BK - 1:
                        wait_blk(1)
                    if p == 2 * BK - 1:
                        wait_blk(0)
                    wait_gather(rbuf)
                    nbuf, nrow = ((p + 1) // BK) % 2, (p + 1) % BK
                    gather(nbuf, nrow, 1 - rbuf)
                    pltpu.sync_copy(rr[rbuf], acc.at[db[bbuf].at[row]],
                                    add=True)
                    if p == BK - 1:
                        fetch_blk(lax.rem(2 * i + 2, nb0), 0)
                    if p == 2 * BK - 1:
                        fetch_blk(lax.rem(2 * i + 3, nb0), 1)
                return carry

            lax.fori_loop(0, nb0 // 2, body, 0)
            wait_gather(0)
            wait_blk(1)

        @pl.when(c == 1)
        def _serialized():
            fetch_blk(0, 0)
            fetch_blk(1, 1)

            def body(i, carry):
                for q in range(2):
                    wait_blk(q)
                    for r in range(BK):
                        pltpu.async_copy(y_hbm.at[sb[q].at[r]], r0,
                                         semg0).wait()
                        pltpu.sync_copy(r0, acc.at[db[q].at[r]], add=True)
                    fetch_blk(lax.rem(2 * i + 2 + q, nb1), q)
                return carry

            lax.fori_loop(0, nb1 // 2, body, 0)
            wait_blk(0)
            wait_blk(1)

        plsc.subcore_barrier()
        pltpu.sync_copy(acc.at[pl.ds(base, rpt)],
                        out_hbm.at[c, pl.ds(base, rpt)])

    return prop_kernel(y, srcs4, dsts4, zblk)


def _tc_prep(deg2, x, W1, Wr1):
    n = x.shape[0]
    h = W1.shape[1]

    def body(deg_ref, x_ref, w_ref, wr_ref, y_ref):
        deg = deg_ref[0, :n, 0:1] + deg_ref[1, :n, 0:1] + 1.0
        dis = lax.rsqrt(deg)
        w = 0.95 * w_ref[...] + 0.05 * wr_ref[...]
        y_ref[...] = jnp.dot(x_ref[...], w,
                             preferred_element_type=jnp.float32) * dis

    return pl.pallas_call(
        body, out_shape=jax.ShapeDtypeStruct((n, h), jnp.float32),
    )(deg2, x, W1, Wr1)


def _tc_mid(acc2, y_prev, deg2, bc, br, Wc, Wr):
    n, h = y_prev.shape

    def body(acc_ref, y_ref, deg_ref, bc_ref, br_ref, wc_ref, wr_ref, o_ref):
        deg = deg_ref[0, :n, 0:1] + deg_ref[1, :n, 0:1] + 1.0
        dis = lax.rsqrt(deg)
        b = 0.95 * bc_ref[...] + 0.05 * br_ref[...]
        tot = acc_ref[0, :n, :] + acc_ref[1, :n, :] + y_ref[...]
        hh = jnp.maximum(tot * dis + b, 0.0)
        w = 0.95 * wc_ref[...] + 0.05 * wr_ref[...]
        o_ref[...] = jnp.dot(hh, w, preferred_element_type=jnp.float32) * dis

    return pl.pallas_call(
        body, out_shape=jax.ShapeDtypeStruct((n, h), jnp.float32),
    )(acc2, y_prev, deg2, bc, br, Wc, Wr)


def _tc_final(acc2, y3, deg2, bc, br, batch2d, Wl1, bl1, Wl2, bl2):
    n, h = y3.shape
    c_out = Wl2.shape[1]

    def body(acc_ref, y_ref, deg_ref, bc_ref, br_ref, bat_ref,
             wl1_ref, bl1_ref, wl2_ref, bl2_ref, o_ref):
        deg = deg_ref[0, :n, 0:1] + deg_ref[1, :n, 0:1] + 1.0
        dis = lax.rsqrt(deg)
        b = 0.95 * bc_ref[...] + 0.05 * br_ref[...]
        tot = acc_ref[0, :n, :] + acc_ref[1, :n, :] + y_ref[...]
        hh = jnp.maximum(tot * dis + b, 0.0)
        gid = lax.broadcasted_iota(jnp.int32, (G, n), 0)
        onehot = jnp.where(gid == jnp.broadcast_to(bat_ref[...], (G, n)),
                           1.0, 0.0)
        sums = jnp.dot(onehot, hh, preferred_element_type=jnp.float32)
        counts = jnp.sum(onehot, axis=1, keepdims=True)
        pooled = sums / jnp.maximum(counts, 1.0)
        z = jnp.maximum(
            jnp.dot(pooled, wl1_ref[...],
                    preferred_element_type=jnp.float32) + bl1_ref[...], 0.0)
        z = jnp.dot(z, wl2_ref[...],
                    preferred_element_type=jnp.float32) + bl2_ref[...]
        m = jnp.max(z, axis=1, keepdims=True)
        lse = jnp.log(jnp.sum(jnp.exp(z - m), axis=1, keepdims=True)) + m
        o_ref[...] = z - lse

    return pl.pallas_call(
        body, out_shape=jax.ShapeDtypeStruct((G, c_out), jnp.float32),
    )(acc2, y3, deg2, bc, br, batch2d, Wl1, bl1, Wl2, bl2)


def kernel(x, edge_index, batch, W1, b1, Wr1, br1, Wc0, bc0, Wc1, bc1,
           Wr, br, Wl1, bl1, Wl2, bl2):
    n, d = x.shape
    e = edge_index.shape[1]
    h = W1.shape[1]

    rpt = -(-n // (NS * CH)) * CH
    acc_rows = NS * rpt
    step = 2 * BK
    pair = -(-e // (NS * CH * 2 * step)) * 2 * step
    t0 = max(step, int(round(pair * F0 / step)) * step)
    t1 = pair - t0
    nb0, nb1 = t0 // BK, t1 // BK
    nbmax = max(nb0, nb1)
    e_pad = NS * pair * CH - e

    srcs = edge_index[0]
    dsts = edge_index[1]
    if e_pad:
        srcs = jnp.concatenate([srcs, jnp.zeros((e_pad,), jnp.int32)])
        dsts = jnp.concatenate([dsts, jnp.full((e_pad,), n, jnp.int32)])

    def to_blocks(flat):
        width = nbmax * BK * CH
        p0 = flat[:NS * t0 * CH].reshape(NS, t0 * CH)
        p1 = flat[NS * t0 * CH:].reshape(NS, t1 * CH)
        p0 = jnp.pad(p0, ((0, 0), (0, width - t0 * CH)))
        p1 = jnp.pad(p1, ((0, 0), (0, width - t1 * CH)))
        return jnp.stack([p0, p1], axis=1).reshape(NW, nbmax, BK, CH)

    srcs4 = to_blocks(srcs)
    dsts4 = to_blocks(dsts)

    zblk = jnp.zeros((CH, d), jnp.float32)
    consts16 = jnp.stack([jnp.zeros((CH, 16), jnp.float32),
                          jnp.ones((CH, 16), jnp.float32)])
    batch2d = batch.reshape(1, n)
    b1r, br1r = b1.reshape(1, h), br1.reshape(1, h)
    bc0r, bc1r, brr = bc0.reshape(1, h), bc1.reshape(1, h), br.reshape(1, h)
    bl1r = bl1.reshape(1, h)
    bl2r = bl2.reshape(1, Wl2.shape[1])

    deg2 = _sc_degree(dsts4, consts16, acc_rows, nb0, nb1)
    y1 = _tc_prep(deg2, x, W1, Wr1)
    acc = _sc_propagate(y1, srcs4, dsts4, zblk, acc_rows, nb0, nb1)
    y2 = _tc_mid(acc, y1, deg2, b1r, br1r, Wc0, Wr)
    acc = _sc_propagate(y2, srcs4, dsts4, zblk, acc_rows, nb0, nb1)
    y3 = _tc_mid(acc, y2, deg2, bc0r, brr, Wc1, Wr)
    acc = _sc_propagate(y3, srcs4, dsts4, zblk, acc_rows, nb0, nb1)
    return _tc_final(acc, y3, deg2, bc1r, brr, batch2d, Wl1, bl1r, Wl2, bl2r)


# speedup vs baseline: 1.0619x; 1.0619x over previous
import functools

import jax
import jax.numpy as jnp
from jax import lax
from jax.experimental import pallas as pl
from jax.experimental.pallas import tpu as pltpu
from jax.experimental.pallas import tpu_sc as plsc

NC = 2
NS = 16
NW = NC * NS
CH = 128
BK = 8
G = 64
F0 = 0.8


def _sc_degree(dsts4, consts16, acc_rows, nb0, nb1):
    nbmax = dsts4.shape[1]
    rpt = acc_rows // NS
    zrep = rpt // CH
    mesh = plsc.VectorSubcoreMesh(core_axis_name="c", subcore_axis_name="s")

    @functools.partial(
        pl.kernel,
        out_type=jax.ShapeDtypeStruct((NC, acc_rows, 16), jnp.float32),
        mesh=mesh,
        scratch_types=[
            pltpu.VMEM((nbmax, BK, CH), jnp.int32),
            pltpu.VMEM((CH, 16), jnp.float32),
            pltpu.VMEM((CH, 16), jnp.float32),
            pltpu.VMEM_SHARED((acc_rows, 16), jnp.float32),
        ],
    )
    def deg_kernel(dsts_hbm, consts_hbm, out_hbm, idx_d, zv, ov, acc):
        c = lax.axis_index("c")
        s = lax.axis_index("s")
        w = s * NC + c
        pltpu.sync_copy(dsts_hbm.at[w], idx_d)
        pltpu.sync_copy(consts_hbm.at[0], zv)
        pltpu.sync_copy(consts_hbm.at[1], ov)
        base = s * rpt
        for r in range(zrep):
            pltpu.sync_copy(zv, acc.at[pl.ds(base + r * CH, CH)])
        plsc.subcore_barrier()

        def body(b, carry):
            for r in range(BK):
                pltpu.sync_copy(ov, acc.at[idx_d.at[b, r]], add=True)
            return carry

        lax.fori_loop(0, lax.select(c == 0, nb0, nb1), body, 0)
        plsc.subcore_barrier()
        pltpu.sync_copy(acc.at[pl.ds(base, rpt)],
                        out_hbm.at[c, pl.ds(base, rpt)])

    return deg_kernel(dsts4, consts16)


def _sc_propagate(y, srcs4, dsts4, zblk, acc_rows, nb0, nb1):
    d = y.shape[1]
    rpt = acc_rows // NS
    zrep = rpt // CH
    mesh = plsc.VectorSubcoreMesh(core_axis_name="c", subcore_axis_name="s")
    assert nb0 % 2 == 0 and nb1 % 2 == 0

    @functools.partial(
        pl.kernel,
        out_type=jax.ShapeDtypeStruct((NC, acc_rows, d), jnp.float32),
        mesh=mesh,
        scratch_types=[
            pltpu.VMEM((BK, CH), jnp.int32),
            pltpu.VMEM((BK, CH), jnp.int32),
            pltpu.VMEM((BK, CH), jnp.int32),
            pltpu.VMEM((BK, CH), jnp.int32),
            pltpu.VMEM((CH, d), jnp.float32),
            pltpu.VMEM((CH, d), jnp.float32),
            pltpu.VMEM_SHARED((acc_rows, d), jnp.float32),
            pltpu.SemaphoreType.DMA,
            pltpu.SemaphoreType.DMA,
            pltpu.SemaphoreType.DMA,
            pltpu.SemaphoreType.DMA,
        ],
    )
    def prop_kernel(y_hbm, srcs_hbm, dsts_hbm, zblk_hbm, out_hbm,
                    sb0, db0, sb1, db1, r0, r1, acc,
                    semi0, semi1, semg0, semg1):
        c = lax.axis_index("c")
        s = lax.axis_index("s")
        w = s * NC + c
        sb = (sb0, sb1)
        db = (db0, db1)
        rr = (r0, r1)
        semi = (semi0, semi1)
        semg = (semg0, semg1)

        pltpu.sync_copy(zblk_hbm, r0)
        base = s * rpt
        for r in range(zrep):
            pltpu.sync_copy(r0, acc.at[pl.ds(base + r * CH, CH)])
        plsc.subcore_barrier()

        def fetch_blk(bidx, buf):
            pltpu.async_copy(srcs_hbm.at[w, bidx], sb[buf], semi[buf])
            pltpu.async_copy(dsts_hbm.at[w, bidx], db[buf], semi[buf])

        def wait_blk(buf):
            pltpu.make_async_copy(srcs_hbm.at[w, 0], sb[buf], semi[buf]).wait()
            pltpu.make_async_copy(dsts_hbm.at[w, 0], db[buf], semi[buf]).wait()

        def gather(bbuf, row, rbuf):
            pltpu.async_copy(y_hbm.at[sb[bbuf].at[row]], rr[rbuf], semg[rbuf])

        def wait_gather(rbuf):
            pltpu.make_async_copy(y_hbm.at[sb0.at[0]], rr[rbuf],
                                  semg[rbuf]).wait()

        @pl.when(c == 0)
        def _pipelined():
            fetch_blk(0, 0)
            fetch_blk(1, 1)
            wait_blk(0)
            gather(0, 0, 0)

            def body(i, carry):
                for p in range(2 * BK):
                    bbuf, row, rbuf = (p // BK) % 2, p % BK, p % 2
                    if p == BK - 1:
                        wait_blk(1)
                    if p == 2 * BK - 1:
                        wait_blk(0)
                    wait_gather(rbuf)
                    nbuf, nrow = ((p + 1) // BK) % 2, (p + 1) % BK
                    gather(nbuf, nrow, 1 - rbuf)
                    pltpu.sync_copy(rr[rbuf], acc.at[db[bbuf].at[row]],
                                    add=True)
                    if p == BK - 1:
                        fetch_blk(lax.rem(2 * i + 2, nb0), 0)
                    if p == 2 * BK - 1:
                        fetch_blk(lax.rem(2 * i + 3, nb0), 1)
                return carry

            lax.fori_loop(0, nb0 // 2, body, 0)
            wait_gather(0)
            wait_blk(1)

        @pl.when(c == 1)
        def _serialized():
            fetch_blk(0, 0)
            fetch_blk(1, 1)

            def body(i, carry):
                for q in range(2):
                    wait_blk(q)
                    for r in range(BK):
                        pltpu.async_copy(y_hbm.at[sb[q].at[r]], r0,
                                         semg0).wait()
                        pltpu.sync_copy(r0, acc.at[db[q].at[r]], add=True)
                    fetch_blk(lax.rem(2 * i + 2 + q, nb1), q)
                return carry

            lax.fori_loop(0, nb1 // 2, body, 0)
            wait_blk(0)
            wait_blk(1)

        plsc.subcore_barrier()
        pltpu.sync_copy(acc.at[pl.ds(base, rpt)],
                        out_hbm.at[c, pl.ds(base, rpt)])

    return prop_kernel(y, srcs4, dsts4, zblk)


def _tc_prep(deg2, x, W1, Wr1):
    n = x.shape[0]
    h = W1.shape[1]

    def body(deg_ref, x_ref, w_ref, wr_ref, y_ref):
        deg = deg_ref[0, :n, 0:1] + deg_ref[1, :n, 0:1] + 1.0
        dis = lax.rsqrt(deg)
        w = 0.95 * w_ref[...] + 0.05 * wr_ref[...]
        y_ref[...] = jnp.dot(x_ref[...], w,
                             preferred_element_type=jnp.float32) * dis

    return pl.pallas_call(
        body, out_shape=jax.ShapeDtypeStruct((n, h), jnp.float32),
    )(deg2, x, W1, Wr1)


def _tc_mid(acc2, y_prev, deg2, bc, br, Wc, Wr):
    n, h = y_prev.shape

    def body(acc_ref, y_ref, deg_ref, bc_ref, br_ref, wc_ref, wr_ref, o_ref):
        deg = deg_ref[0, :n, 0:1] + deg_ref[1, :n, 0:1] + 1.0
        dis = lax.rsqrt(deg)
        b = 0.95 * bc_ref[...] + 0.05 * br_ref[...]
        tot = acc_ref[0, :n, :] + acc_ref[1, :n, :] + y_ref[...]
        hh = jnp.maximum(tot * dis + b, 0.0)
        w = 0.95 * wc_ref[...] + 0.05 * wr_ref[...]
        o_ref[...] = jnp.dot(hh, w, preferred_element_type=jnp.float32) * dis

    return pl.pallas_call(
        body, out_shape=jax.ShapeDtypeStruct((n, h), jnp.float32),
    )(acc2, y_prev, deg2, bc, br, Wc, Wr)


def _tc_final(acc2, y3, deg2, bc, br, batch2d, Wl1, bl1, Wl2, bl2):
    n, h = y3.shape
    c_out = Wl2.shape[1]

    def body(acc_ref, y_ref, deg_ref, bc_ref, br_ref, bat_ref,
             wl1_ref, bl1_ref, wl2_ref, bl2_ref, o_ref):
        deg = deg_ref[0, :n, 0:1] + deg_ref[1, :n, 0:1] + 1.0
        dis = lax.rsqrt(deg)
        b = 0.95 * bc_ref[...] + 0.05 * br_ref[...]
        tot = acc_ref[0, :n, :] + acc_ref[1, :n, :] + y_ref[...]
        hh = jnp.maximum(tot * dis + b, 0.0)
        gid = lax.broadcasted_iota(jnp.int32, (G, n), 0)
        onehot = jnp.where(gid == jnp.broadcast_to(bat_ref[...], (G, n)),
                           1.0, 0.0)
        sums = jnp.dot(onehot, hh, preferred_element_type=jnp.float32)
        counts = jnp.sum(onehot, axis=1, keepdims=True)
        pooled = sums / jnp.maximum(counts, 1.0)
        z = jnp.maximum(
            jnp.dot(pooled, wl1_ref[...],
                    preferred_element_type=jnp.float32) + bl1_ref[...], 0.0)
        z = jnp.dot(z, wl2_ref[...],
                    preferred_element_type=jnp.float32) + bl2_ref[...]
        m = jnp.max(z, axis=1, keepdims=True)
        lse = jnp.log(jnp.sum(jnp.exp(z - m), axis=1, keepdims=True)) + m
        o_ref[...] = z - lse

    return pl.pallas_call(
        body, out_shape=jax.ShapeDtypeStruct((G, c_out), jnp.float32),
    )(acc2, y3, deg2, bc, br, batch2d, Wl1, bl1, Wl2, bl2)


def kernel(x, edge_index, batch, W1, b1, Wr1, br1, Wc0, bc0, Wc1, bc1,
           Wr, br, Wl1, bl1, Wl2, bl2):
    n, d = x.shape
    e = edge_index.shape[1]
    h = W1.shape[1]

    rpt = -(-n // (NS * CH)) * CH
    acc_rows = NS * rpt
    step = 2 * BK
    pair = -(-e // (NS * CH * 2 * step)) * 2 * step
    t0 = max(step, int(round(pair * F0 / step)) * step)
    t1 = pair - t0
    nb0, nb1 = t0 // BK, t1 // BK
    nbmax = max(nb0, nb1)
    e_pad = NS * pair * CH - e

    srcs = edge_index[0]
    dsts = edge_index[1]
    if e_pad:
        srcs = jnp.concatenate([srcs, jnp.zeros((e_pad,), jnp.int32)])
        dsts = jnp.concatenate([dsts, jnp.full((e_pad,), n, jnp.int32)])

    def to_blocks(flat):
        width = nbmax * BK * CH
        p0 = flat[:NS * t0 * CH].reshape(NS, t0 * CH)
        p1 = flat[NS * t0 * CH:].reshape(NS, t1 * CH)
        p0 = jnp.pad(p0, ((0, 0), (0, width - t0 * CH)))
        p1 = jnp.pad(p1, ((0, 0), (0, width - t1 * CH)))
        return jnp.stack([p0, p1], axis=1).reshape(NW, nbmax, BK, CH)

    srcs4 = to_blocks(srcs)
    dsts4 = to_blocks(dsts)

    zblk = jnp.zeros((CH, d), jnp.float32)
    consts16 = jnp.stack([jnp.zeros((CH, 16), jnp.float32),
                          jnp.ones((CH, 16), jnp.float32)])
    batch2d = batch.reshape(1, n)
    b1r, br1r = b1.reshape(1, h), br1.reshape(1, h)
    bc0r, bc1r, brr = bc0.reshape(1, h), bc1.reshape(1, h), br.reshape(1, h)
    bl1r = bl1.reshape(1, h)
    bl2r = bl2.reshape(1, Wl2.shape[1])

    deg2 = _sc_degree(dsts4, consts16, acc_rows, nb0, nb1)
    y1 = _tc_prep(deg2, x, W1, Wr1)
    acc = _sc_propagate(y1, srcs4, dsts4, zblk, acc_rows, nb0, nb1)
    y2 = _tc_mid(acc, y1, deg2, b1r, br1r, Wc0, Wr)
    acc = _sc_propagate(y2, srcs4, dsts4, zblk, acc_rows, nb0, nb1)
    y3 = _tc_mid(acc, y2, deg2, bc0r, brr, Wc1, Wr)
    acc = _sc_propagate(y3, srcs4, dsts4, zblk, acc_rows, nb0, nb1)
    return _tc_final(acc, y3, deg2, bc1r, brr, batch2d, Wl1, bl1r, Wl2, bl2r)
